# baseline (speedup 1.0000x reference)
import functools

import jax
import jax.numpy as jnp
from jax import lax
from jax.experimental import pallas as pl
from jax.experimental.pallas import tpu as pltpu
from jax.experimental.pallas import tpu_sc as plsc

_LANES = 16
_NC = 2
_NS = 16
_NW = _NC * _NS
_BATCH = 16384
_DIM = 64
_BPW = _BATCH // _NW
_GROUPS = _BPW // _LANES

_mesh = plsc.VectorSubcoreMesh(core_axis_name="c", subcore_axis_name="s")


@functools.partial(
    pl.kernel,
    mesh=_mesh,
    out_type=jax.ShapeDtypeStruct((_BATCH,), jnp.float32),
    scratch_types=[
        pltpu.VMEM((_BPW,), jnp.int32),
        pltpu.VMEM((_BPW,), jnp.int32),
        pltpu.VMEM((_BPW, _DIM), jnp.float32),
        pltpu.VMEM((_BPW, _DIM), jnp.float32),
        pltpu.VMEM((_BPW,), jnp.float32),
        pltpu.SemaphoreType.DMA,
        pltpu.SemaphoreType.DMA,
    ],
    compiler_params=pltpu.CompilerParams(
        needs_layout_passes=False, use_tc_tiling_on_sc=False),
)
def _sc_dot(uidx_hbm, pidx_hbm, user_hbm, place_hbm, out_hbm,
            uidx_v, pidx_v, urows_v, prows_v, out_v, usem, psem):
    wid = lax.axis_index("s") * _NC + lax.axis_index("c")
    base = wid * _BPW

    pltpu.sync_copy(uidx_hbm.at[pl.ds(base, _BPW)], uidx_v)
    pltpu.sync_copy(pidx_hbm.at[pl.ds(base, _BPW)], pidx_v)
    cu = pltpu.async_copy(user_hbm.at[uidx_v], urows_v, usem)
    cp = pltpu.async_copy(place_hbm.at[pidx_v], prows_v, psem)
    cu.wait()
    cp.wait()

    def group_body(g, carry):
        rows = g * _LANES + lax.iota(jnp.int32, _LANES)

        def d_body(dd, acc):
            for k in range(4):
                col = jnp.full((_LANES,), dd * 4 + k, jnp.int32)
                u = plsc.load_gather(urows_v, [rows, col])
                p = plsc.load_gather(prows_v, [rows, col])
                acc = acc + u * p
            return acc

        acc = lax.fori_loop(0, _DIM // 4, d_body,
                            jnp.zeros((_LANES,), jnp.float32))
        out_v[pl.ds(g * _LANES, _LANES)] = acc
        return carry

    lax.fori_loop(0, _GROUPS, group_body, 0)
    pltpu.sync_copy(out_v, out_hbm.at[pl.ds(base, _BPW)])


def kernel(inputs, user_table, place_table):
    uidx = inputs[:, 0]
    pidx = inputs[:, 1]
    return _sc_dot(uidx, pidx, user_table, place_table)
